# baseline (speedup 1.0000x reference)
import functools

import jax
import jax.numpy as jnp
from jax import lax
from jax.experimental import pallas as pl
from jax.experimental.pallas import tpu as pltpu
from jax.experimental.pallas import tpu_sc as plsc

B, S, D = 4, 2048, 1024
V, H = 128, 1024
TOTAL = S + V
NW = 32
WPB = NW // B
CHUNK = TOTAL // WPB
ZPAD = CHUNK + 8
AUXR = 8 + V + ZPAD
CAP = 32
NB = 3


def _mlp_body(seq_ref, w_emb_ref, w1_ref, b1_ref, w2_ref, b2_ref, aux_ref):
    h = jnp.tanh(
        jnp.dot(w_emb_ref[...], w1_ref[...], preferred_element_type=jnp.float32)
        + b1_ref[...]
    )
    virt = (
        jnp.dot(h, w2_ref[...], preferred_element_type=jnp.float32) + b2_ref[...]
    )
    aux_ref[...] = jnp.zeros((B, AUXR, D), jnp.float32)
    for b in range(B):
        rb = seq_ref[b] % 8
        for s in range(8):
            @pl.when(rb == s)
            def _():
                aux_ref[b, s:s + V, :] = virt


def _make_aux(seq32, w_emb, w1, b1, w2, b2):
    return pl.pallas_call(
        _mlp_body,
        out_shape=jax.ShapeDtypeStruct((B, AUXR, D), jnp.float32),
        in_specs=[
            pl.BlockSpec(memory_space=pltpu.SMEM),
            pl.BlockSpec(memory_space=pltpu.VMEM),
            pl.BlockSpec(memory_space=pltpu.VMEM),
            pl.BlockSpec(memory_space=pltpu.VMEM),
            pl.BlockSpec(memory_space=pltpu.VMEM),
            pl.BlockSpec(memory_space=pltpu.VMEM),
        ],
    )(seq32, w_emb, w1, b1.reshape(1, H), w2, b2.reshape(1, D))


def _al8(x):
    return pl.multiple_of(x, 8)


def _piece(src_ref, dst_ref, buf, src_base, dst_base, off, rows):
    pltpu.sync_copy(src_ref.at[pl.ds(_al8(src_base + off), rows)], buf.at[pl.ds(0, rows)])
    pltpu.sync_copy(buf.at[pl.ds(0, rows)], dst_ref.at[pl.ds(_al8(dst_base + off), rows)])


def _copy_region(src_ref, dst_ref, bufs, semg, sems, src_base, dst_base, count,
                 max_count):
    nblk = max_count // CAP

    def fired(i):
        return count >= (i + 1) * CAP

    def launch(i):
        j = i % NB

        @pl.when(fired(i))
        def _():
            if i >= NB:
                pltpu.make_async_copy(src_ref.at[pl.ds(0, CAP)], bufs[j], sems[j]).wait()
            pltpu.async_copy(src_ref.at[pl.ds(_al8(src_base + i * CAP), CAP)],
                             bufs[j], semg[j])

    def retire(k):
        j = k % NB

        @pl.when(fired(k))
        def _():
            pltpu.make_async_copy(src_ref.at[pl.ds(0, CAP)], bufs[j], semg[j]).wait()
            pltpu.async_copy(bufs[j], dst_ref.at[pl.ds(_al8(dst_base + k * CAP), CAP)],
                             sems[j])

    for i in range(nblk):
        launch(i)
        if i - (NB - 1) >= 0:
            retire(i - (NB - 1))
    for k in range(max(0, nblk - NB + 1), nblk):
        retire(k)
    for k in range(nblk):
        @pl.when(fired(k) & jnp.logical_not(fired(k + NB)))
        def _():
            pltpu.make_async_copy(src_ref.at[pl.ds(0, CAP)], bufs[k % NB],
                                  sems[k % NB]).wait()

    blkrows = (count // CAP) * CAP
    rem = count - blkrows
    for p in (16, 8):
        if p >= CAP or p > max_count:
            continue
        off = blkrows + jnp.bitwise_and(rem, jnp.int32(~(2 * p - 1)))

        @pl.when(jnp.bitwise_and(rem, jnp.int32(p)) != 0)
        def _():
            _piece(src_ref, dst_ref, bufs[0], src_base, dst_base, off, p)


@functools.partial(
    pl.kernel,
    mesh=plsc.VectorSubcoreMesh(core_axis_name="c", subcore_axis_name="s"),
    out_type=jax.ShapeDtypeStruct((B * TOTAL, D), jnp.float32),
    scratch_types=[
        pltpu.VMEM((32,), jnp.int32),
        pltpu.VMEM((CAP, D), jnp.float32),
        pltpu.VMEM((CAP, D), jnp.float32),
        pltpu.VMEM((CAP, D), jnp.float32),
        pltpu.VMEM((8, D), jnp.float32),
        pltpu.VMEM((8, D), jnp.float32),
        pltpu.SemaphoreType.DMA,
        pltpu.SemaphoreType.DMA,
        pltpu.SemaphoreType.DMA,
        pltpu.SemaphoreType.DMA,
        pltpu.SemaphoreType.DMA,
        pltpu.SemaphoreType.DMA,
    ],
)
def _sc_union(inp_hbm, seq_hbm, aux_hbm, out_hbm, seq_v, buf_a, buf_b, buf_c,
              bin8, baux8, semg0, semg1, semg2, sems0, sems1, sems2):
    bufs = (buf_a, buf_b, buf_c)
    semg = (semg0, semg1, semg2)
    sems = (sems0, sems1, sems2)
    cid = lax.axis_index("c")
    sid = lax.axis_index("s")
    w = sid * 2 + cid
    b = w // WPB
    t0 = (w % WPB) * CHUNK

    pltpu.sync_copy(seq_hbm, seq_v)
    ln = seq_v[pl.ds(b, 16)][0]
    r = ln % 8
    g = ln - r
    g2 = g + V
    out_base = b * TOTAL

    k1 = jnp.clip(g - t0, 0, CHUNK)
    _copy_region(inp_hbm, out_hbm, bufs, semg, sems, b * S + t0, out_base + t0,
                 k1, CHUNK)

    va = g + jnp.where(r > 0, 8, 0)
    s2 = jnp.clip(va, t0, t0 + CHUNK)
    e2 = jnp.clip(g2, t0, t0 + CHUNK)
    _copy_region(aux_hbm.at[b], out_hbm, bufs, semg, sems, s2 - g,
                 out_base + s2, e2 - s2, V)

    @pl.when((r > 0) & (g2 >= t0) & (g2 < t0 + CHUNK))
    def _():
        _piece(aux_hbm.at[b], out_hbm, bufs[0], jnp.int32(V), out_base + g2, 0, 8)

    z0 = jnp.clip(g2 + jnp.where(r > 0, 8, 0), t0, t0 + CHUNK)
    _copy_region(aux_hbm.at[b], out_hbm, bufs, semg, sems, jnp.int32(V + 8),
                 out_base + z0, t0 + CHUNK - z0, CHUNK)

    @pl.when((r > 0) & (g >= t0) & (g < t0 + CHUNK))
    def _():
        pltpu.sync_copy(inp_hbm.at[pl.ds(_al8(b * S + g), 8)], bin8)
        pltpu.sync_copy(aux_hbm.at[b, pl.ds(0, 8)], baux8)

        def body(i, _):
            jj = i // 64
            c = (i % 64) * 16
            baux8[jj, pl.ds(c, 16)] = bin8[jj, pl.ds(c, 16)]
            return 0

        lax.fori_loop(0, r * 64, body, 0)
        pltpu.sync_copy(baux8, out_hbm.at[pl.ds(_al8(out_base + g), 8)])


def kernel(inputs, seq_len, W_emb, W1, b1, W2, b2):
    seq32 = seq_len.astype(jnp.int32)
    aux = _make_aux(seq32, W_emb, W1, b1, W2, b2)
    seq_pad = jnp.zeros((32,), jnp.int32).at[:B].set(seq32)
    out2d = _sc_union(inputs.reshape(B * S, D), seq_pad, aux)
    return out2d.reshape(B, TOTAL, D), seq_len + V


# speedup vs baseline: 1.0428x; 1.0428x over previous
import functools

import jax
import jax.numpy as jnp
from jax import lax
from jax.experimental import pallas as pl
from jax.experimental.pallas import tpu as pltpu
from jax.experimental.pallas import tpu_sc as plsc

B, S, D = 4, 2048, 1024
V, H = 128, 1024
TOTAL = S + V
NW = 32
WPB = NW // B
CHUNK = TOTAL // WPB
ZPAD = CHUNK + 8
AUXR = 8 + V + ZPAD
CAP = 104


def _mlp_body(seq_ref, w_emb_ref, w1_ref, b1_ref, w2_ref, b2_ref, aux_ref):
    h = jnp.tanh(
        jnp.dot(w_emb_ref[...], w1_ref[...], preferred_element_type=jnp.float32)
        + b1_ref[...]
    )
    virt = (
        jnp.dot(h, w2_ref[...], preferred_element_type=jnp.float32) + b2_ref[...]
    )
    aux_ref[...] = jnp.zeros((B, AUXR, D), jnp.float32)
    for b in range(B):
        rb = seq_ref[b] % 8
        for s in range(8):
            @pl.when(rb == s)
            def _():
                aux_ref[b, s:s + V, :] = virt


def _make_aux(seq32, w_emb, w1, b1, w2, b2):
    return pl.pallas_call(
        _mlp_body,
        out_shape=jax.ShapeDtypeStruct((B, AUXR, D), jnp.float32),
        in_specs=[
            pl.BlockSpec(memory_space=pltpu.SMEM),
            pl.BlockSpec(memory_space=pltpu.VMEM),
            pl.BlockSpec(memory_space=pltpu.VMEM),
            pl.BlockSpec(memory_space=pltpu.VMEM),
            pl.BlockSpec(memory_space=pltpu.VMEM),
            pl.BlockSpec(memory_space=pltpu.VMEM),
        ],
    )(seq32, w_emb, w1, b1.reshape(1, H), w2, b2.reshape(1, D))


def _al8(x):
    return pl.multiple_of(x, 8)


def _piece(src_ref, dst_ref, buf, src_base, dst_base, off, rows):
    pltpu.sync_copy(src_ref.at[pl.ds(_al8(src_base + off), rows)], buf.at[pl.ds(0, rows)])
    pltpu.sync_copy(buf.at[pl.ds(0, rows)], dst_ref.at[pl.ds(_al8(dst_base + off), rows)])


def _copy_region(src_ref, dst_ref, buf, src_base, dst_base, count, max_count):
    nblk = max_count // CAP
    for i in range(nblk):
        @pl.when(count >= (i + 1) * CAP)
        def _():
            _piece(src_ref, dst_ref, buf, src_base, dst_base, i * CAP, CAP)
    blkrows = (count // CAP) * CAP
    rem = count - blkrows
    for p in (64, 32, 16, 8):
        if p >= CAP or p > max_count:
            continue
        off = blkrows + jnp.bitwise_and(rem, jnp.int32(~(2 * p - 1)))

        @pl.when(jnp.bitwise_and(rem, jnp.int32(p)) != 0)
        def _():
            _piece(src_ref, dst_ref, buf, src_base, dst_base, off, p)


@functools.partial(
    pl.kernel,
    mesh=plsc.VectorSubcoreMesh(core_axis_name="c", subcore_axis_name="s"),
    out_type=jax.ShapeDtypeStruct((B * TOTAL, D), jnp.float32),
    scratch_types=[
        pltpu.VMEM((32,), jnp.int32),
        pltpu.VMEM((CAP, D), jnp.float32),
        pltpu.VMEM((8, D), jnp.float32),
        pltpu.VMEM((8, D), jnp.float32),
    ],
)
def _sc_union(inp_hbm, seq_hbm, aux_hbm, out_hbm, seq_v, buf, bin8, baux8):
    cid = lax.axis_index("c")
    sid = lax.axis_index("s")
    w = sid * 2 + cid
    b = w // WPB
    t0 = (w % WPB) * CHUNK

    pltpu.sync_copy(seq_hbm, seq_v)
    ln = seq_v[pl.ds(b, 16)][0]
    r = ln % 8
    g = ln - r
    g2 = g + V
    out_base = b * TOTAL

    k1 = jnp.clip(g - t0, 0, CHUNK)
    _copy_region(inp_hbm, out_hbm, buf, b * S + t0, out_base + t0, k1, CHUNK)

    va = g + jnp.where(r > 0, 8, 0)
    s2 = jnp.clip(va, t0, t0 + CHUNK)
    e2 = jnp.clip(g2, t0, t0 + CHUNK)
    _copy_region(aux_hbm.at[b], out_hbm, buf, s2 - g, out_base + s2, e2 - s2, V)

    @pl.when((r > 0) & (g2 >= t0) & (g2 < t0 + CHUNK))
    def _():
        _piece(aux_hbm.at[b], out_hbm, buf, jnp.int32(V), out_base + g2, 0, 8)

    z0 = jnp.clip(g2 + jnp.where(r > 0, 8, 0), t0, t0 + CHUNK)
    _copy_region(aux_hbm.at[b], out_hbm, buf, jnp.int32(V + 8), out_base + z0,
                 t0 + CHUNK - z0, CHUNK)

    @pl.when((r > 0) & (g >= t0) & (g < t0 + CHUNK))
    def _():
        pltpu.sync_copy(inp_hbm.at[pl.ds(_al8(b * S + g), 8)], bin8)
        pltpu.sync_copy(aux_hbm.at[b, pl.ds(0, 8)], baux8)

        def body(i, _):
            jj = i // 64
            c = (i % 64) * 16
            baux8[jj, pl.ds(c, 16)] = bin8[jj, pl.ds(c, 16)]
            return 0

        lax.fori_loop(0, r * 64, body, 0)
        pltpu.sync_copy(baux8, out_hbm.at[pl.ds(_al8(out_base + g), 8)])


def kernel(inputs, seq_len, W_emb, W1, b1, W2, b2):
    seq32 = seq_len.astype(jnp.int32)
    aux = _make_aux(seq32, W_emb, W1, b1, W2, b2)
    seq_pad = jnp.zeros((32,), jnp.int32).at[:B].set(seq32)
    out2d = _sc_union(inputs.reshape(B * S, D), seq_pad, aux)
    return out2d.reshape(B, TOTAL, D), seq_len + V


# speedup vs baseline: 1.0580x; 1.0146x over previous
import functools

import jax
import jax.numpy as jnp
from jax import lax
from jax.experimental import pallas as pl
from jax.experimental.pallas import tpu as pltpu
from jax.experimental.pallas import tpu_sc as plsc

B, S, D = 4, 2048, 1024
V, H = 128, 1024
TOTAL = S + V
NW = 32
WPB = NW // B
CHUNK = TOTAL // WPB
ZPAD = CHUNK + 8
AUXR = 8 + V + ZPAD
CAP = 96


def _mlp_body(seq_ref, w_emb_ref, w1_ref, b1_ref, w2_ref, b2_ref, aux_ref):
    h = jnp.tanh(
        jnp.dot(w_emb_ref[...], w1_ref[...], preferred_element_type=jnp.float32)
        + b1_ref[...]
    )
    virt = (
        jnp.dot(h, w2_ref[...], preferred_element_type=jnp.float32) + b2_ref[...]
    )
    aux_ref[...] = jnp.zeros((B, AUXR, D), jnp.float32)
    for b in range(B):
        rb = seq_ref[b] % 8
        for s in range(8):
            @pl.when(rb == s)
            def _():
                aux_ref[b, s:s + V, :] = virt


def _make_aux(seq32, w_emb, w1, b1, w2, b2):
    return pl.pallas_call(
        _mlp_body,
        out_shape=jax.ShapeDtypeStruct((B, AUXR, D), jnp.float32),
        in_specs=[
            pl.BlockSpec(memory_space=pltpu.SMEM),
            pl.BlockSpec(memory_space=pltpu.VMEM),
            pl.BlockSpec(memory_space=pltpu.VMEM),
            pl.BlockSpec(memory_space=pltpu.VMEM),
            pl.BlockSpec(memory_space=pltpu.VMEM),
            pl.BlockSpec(memory_space=pltpu.VMEM),
        ],
    )(seq32, w_emb, w1, b1.reshape(1, H), w2, b2.reshape(1, D))


def _al8(x):
    return pl.multiple_of(x, 8)


def _piece(src_ref, dst_ref, buf, src_base, dst_base, off, rows):
    pltpu.sync_copy(src_ref.at[pl.ds(_al8(src_base + off), rows)], buf.at[pl.ds(0, rows)])
    pltpu.sync_copy(buf.at[pl.ds(0, rows)], dst_ref.at[pl.ds(_al8(dst_base + off), rows)])


def _copy_region(src_ref, dst_ref, buf, src_base, dst_base, count, max_count):
    nblk = max_count // CAP
    for i in range(nblk):
        @pl.when(count >= (i + 1) * CAP)
        def _():
            _piece(src_ref, dst_ref, buf, src_base, dst_base, i * CAP, CAP)
    blkrows = (count // CAP) * CAP
    rem = count - blkrows
    for p in (64, 32, 16, 8):
        if p >= CAP or p > max_count:
            continue
        off = blkrows + jnp.bitwise_and(rem, jnp.int32(~(2 * p - 1)))

        @pl.when(jnp.bitwise_and(rem, jnp.int32(p)) != 0)
        def _():
            _piece(src_ref, dst_ref, buf, src_base, dst_base, off, p)


@functools.partial(
    pl.kernel,
    mesh=plsc.VectorSubcoreMesh(core_axis_name="c", subcore_axis_name="s"),
    out_type=jax.ShapeDtypeStruct((B * TOTAL, D), jnp.float32),
    scratch_types=[
        pltpu.VMEM((32,), jnp.int32),
        pltpu.VMEM((CAP, D), jnp.float32),
        pltpu.VMEM((8, D), jnp.float32),
        pltpu.VMEM((8, D), jnp.float32),
    ],
)
def _sc_union(inp_hbm, seq_hbm, aux_hbm, out_hbm, seq_v, buf, bin8, baux8):
    cid = lax.axis_index("c")
    sid = lax.axis_index("s")
    w = sid * 2 + cid
    b = w // WPB
    t0 = (w % WPB) * CHUNK

    pltpu.sync_copy(seq_hbm, seq_v)
    ln = seq_v[pl.ds(b, 16)][0]
    r = ln % 8
    g = ln - r
    g2 = g + V
    out_base = b * TOTAL

    k1 = jnp.clip(g - t0, 0, CHUNK)
    _copy_region(inp_hbm, out_hbm, buf, b * S + t0, out_base + t0, k1, CHUNK)

    va = g + jnp.where(r > 0, 8, 0)
    s2 = jnp.clip(va, t0, t0 + CHUNK)
    e2 = jnp.clip(g2, t0, t0 + CHUNK)
    _copy_region(aux_hbm.at[b], out_hbm, buf, s2 - g, out_base + s2, e2 - s2, V)

    @pl.when((r > 0) & (g2 >= t0) & (g2 < t0 + CHUNK))
    def _():
        _piece(aux_hbm.at[b], out_hbm, buf, jnp.int32(V), out_base + g2, 0, 8)

    z0 = jnp.clip(g2 + jnp.where(r > 0, 8, 0), t0, t0 + CHUNK)
    _copy_region(aux_hbm.at[b], out_hbm, buf, jnp.int32(V + 8), out_base + z0,
                 t0 + CHUNK - z0, CHUNK)

    @pl.when((r > 0) & (g >= t0) & (g < t0 + CHUNK))
    def _():
        pltpu.sync_copy(inp_hbm.at[pl.ds(_al8(b * S + g), 8)], bin8)
        pltpu.sync_copy(aux_hbm.at[b, pl.ds(0, 8)], baux8)

        def body(i, _):
            jj = i // 64
            c = (i % 64) * 16
            baux8[jj, pl.ds(c, 16)] = bin8[jj, pl.ds(c, 16)]
            return 0

        lax.fori_loop(0, r * 64, body, 0)
        pltpu.sync_copy(baux8, out_hbm.at[pl.ds(_al8(out_base + g), 8)])


def kernel(inputs, seq_len, W_emb, W1, b1, W2, b2):
    seq32 = seq_len.astype(jnp.int32)
    aux = _make_aux(seq32, W_emb, W1, b1, W2, b2)
    seq_pad = jnp.zeros((32,), jnp.int32).at[:B].set(seq32)
    out2d = _sc_union(inputs.reshape(B * S, D), seq_pad, aux)
    return out2d.reshape(B, TOTAL, D), seq_len + V
